# Initial kernel scaffold; baseline (speedup 1.0000x reference)
import jax
import jax.numpy as jnp
from jax.experimental import pallas as pl


def kernel(x, embeddings, position_encodings):
    raise NotImplementedError("write your pallas kernel here")


# speedup vs baseline: 1.3888x; 1.3888x over previous
import functools

import jax
import jax.numpy as jnp
from jax import lax
from jax.experimental import pallas as pl
from jax.experimental.pallas import tpu as pltpu
from jax.experimental.pallas import tpu_sc as plsc

_LANES = 16
_SUB = 100


@functools.cache
def _make_kernel(B, L, D, V):
    N = B * L
    info = plsc.get_sparse_core_info()
    NC, NS = info.num_cores, info.num_subcores
    NW = NC * NS
    per_w = N // NW
    CH_SUB = 8
    CH = _SUB * CH_SUB
    seq_per_chunk = CH // L
    n_chunks = per_w // CH
    assert per_w % CH == 0 and CH % L == 0 and D == 2 * _LANES

    mesh = plsc.VectorSubcoreMesh(core_axis_name="c", subcore_axis_name="s")

    @functools.partial(
        pl.kernel,
        mesh=mesh,
        out_type=jax.ShapeDtypeStruct((N, D), jnp.float32),
        compiler_params=pltpu.CompilerParams(use_tc_tiling_on_sc=False),
        scratch_types=[
            pltpu.VMEM((CH_SUB, _SUB), jnp.int32),
            pltpu.VMEM((CH, D), jnp.float32),
            pltpu.VMEM((L, D), jnp.float32),
            pltpu.SemaphoreType.DMA,
        ],
    )
    def k(idx_hbm, emb_hbm, pos_hbm, out_hbm, idx_v, rows_v, pos_v, sem):
        wid = lax.axis_index("s") * NC + lax.axis_index("c")
        pltpu.sync_copy(pos_hbm, pos_v)
        base_sub = wid * (per_w // _SUB)

        def chunk_body(c, carry):
            sub0 = base_sub + c * CH_SUB
            row0 = sub0 * _SUB
            pltpu.sync_copy(idx_hbm.at[pl.ds(sub0, CH_SUB)], idx_v)
            copies = [
                pltpu.async_copy(
                    emb_hbm.at[idx_v.at[j]],
                    rows_v.at[pl.ds(j * _SUB, _SUB)],
                    sem,
                )
                for j in range(CH_SUB)
            ]
            for cop in copies:
                cop.wait()

            def add_body(l, c2):
                p0 = pos_v[l, pl.ds(0, _LANES)]
                p1 = pos_v[l, pl.ds(_LANES, _LANES)]
                for s in range(seq_per_chunk):
                    r = s * L + l
                    rows_v[r, pl.ds(0, _LANES)] += p0
                    rows_v[r, pl.ds(_LANES, _LANES)] += p1
                return c2

            lax.fori_loop(0, L, add_body, 0)
            pltpu.sync_copy(rows_v, out_hbm.at[pl.ds(row0, CH)])
            return carry

        lax.fori_loop(0, n_chunks, chunk_body, 0)

    return k


def kernel(x, embeddings, position_encodings):
    B, L = x.shape
    V, D = embeddings.shape
    k = _make_kernel(B, L, D, V)
    idx2d = x.reshape(-1, _SUB)
    out = k(idx2d, embeddings, position_encodings)
    return out.reshape(B, L, D)


# speedup vs baseline: 1.4915x; 1.0739x over previous
import functools

import jax
import jax.numpy as jnp
from jax import lax
from jax.experimental import pallas as pl
from jax.experimental.pallas import tpu as pltpu
from jax.experimental.pallas import tpu_sc as plsc

_LANES = 16
_SUB = 100


@functools.cache
def _make_kernel(B, L, D, V):
    N = B * L
    info = plsc.get_sparse_core_info()
    NC, NS = info.num_cores, info.num_subcores
    NW = NC * NS
    per_w = N // NW
    n_sub = per_w // _SUB
    CH_SUB = 4
    CH = _SUB * CH_SUB
    seq_per_chunk = CH // L
    n_chunks = per_w // CH
    NBUF = 4
    n_outer = n_chunks // NBUF
    assert per_w % CH == 0 and CH % L == 0 and D == 2 * _LANES
    assert n_chunks % NBUF == 0 and NBUF == 4 and n_chunks >= 2 * NBUF

    mesh = plsc.VectorSubcoreMesh(core_axis_name="c", subcore_axis_name="s")

    @functools.partial(
        pl.kernel,
        mesh=mesh,
        out_type=jax.ShapeDtypeStruct((N, D), jnp.float32),
        compiler_params=pltpu.CompilerParams(use_tc_tiling_on_sc=False),
        scratch_types=[
            pltpu.VMEM((n_sub, _SUB), jnp.int32),
            pltpu.VMEM((NBUF, CH, D), jnp.float32),
            pltpu.VMEM((L, D), jnp.float32),
            [pltpu.SemaphoreType.DMA] * NBUF,
            [pltpu.SemaphoreType.DMA] * NBUF,
        ],
    )
    def k(idx_hbm, emb_hbm, pos_hbm, out_hbm, idxw, rows, pos_v, sems_g, sems_o):
        wid = lax.axis_index("s") * NC + lax.axis_index("c")
        pltpu.sync_copy(pos_hbm, pos_v)
        sub_base = wid * n_sub
        row_base = wid * per_w
        pltpu.sync_copy(idx_hbm.at[pl.ds(sub_base, n_sub)], idxw)

        def fire_gather(c, b):
            for j in range(CH_SUB):
                pltpu.async_copy(
                    emb_hbm.at[idxw.at[c * CH_SUB + j]],
                    rows.at[b, pl.ds(j * _SUB, _SUB)],
                    sems_g[b],
                )

        def wait_gather(b):
            for j in range(CH_SUB):
                pltpu.make_async_copy(
                    emb_hbm.at[idxw.at[0]],
                    rows.at[b, pl.ds(j * _SUB, _SUB)],
                    sems_g[b],
                ).wait()

        def drain_out(b):
            pltpu.make_async_copy(
                rows.at[b],
                out_hbm.at[pl.ds(row_base, CH)],
                sems_o[b],
            ).wait()

        fire_gather(0, 0)
        fire_gather(1, 1)

        def outer_body(p, carry):
            for u in range(NBUF):
                c = NBUF * p + u
                bg = (u + 2) % NBUF
                if u < 2:
                    pl.when(p >= 1)(lambda bg=bg: drain_out(bg))
                else:
                    drain_out(bg)
                if u < 2:
                    fire_gather(c + 2, bg)
                else:
                    pl.when(p <= n_outer - 2)(
                        lambda c=c, bg=bg: fire_gather(c + 2, bg)
                    )
                wait_gather(u)

                @plsc.parallel_loop(0, L, unroll=2)
                def add_body(l, _u=u):
                    p0 = pos_v[l, pl.ds(0, _LANES)]
                    p1 = pos_v[l, pl.ds(_LANES, _LANES)]
                    for s in range(seq_per_chunk):
                        r = s * L + l
                        rows[_u, r, pl.ds(0, _LANES)] += p0
                        rows[_u, r, pl.ds(_LANES, _LANES)] += p1

                pltpu.async_copy(
                    rows.at[u],
                    out_hbm.at[pl.ds(row_base + c * CH, CH)],
                    sems_o[u],
                )
            return carry

        lax.fori_loop(0, n_outer, outer_body, 0)
        drain_out(2)
        drain_out(3)

    return k


def kernel(x, embeddings, position_encodings):
    B, L = x.shape
    V, D = embeddings.shape
    k = _make_kernel(B, L, D, V)
    idx2d = x.reshape(-1, _SUB)
    out = k(idx2d, embeddings, position_encodings)
    return out.reshape(B, L, D)
